# baseline (speedup 1.0000x reference)
import jax
import jax.numpy as jnp
from jax.experimental import pallas as pl
from jax.experimental.pallas import tpu as pltpu

N = 4096
F_IN = 512
H = 256
C = 64
EPS = 0.3
NB = 16
BM = N // NB
NE = 4
BE = N // NE
NQ = 8
BQ = N // NQ
S_FA1 = NE
S_FA2 = NE + NB
N_STEPS = NE + NB + NQ


def _mega_body(feature_ref, adj_l_ref, adj_r_ref, lin_w_ref, lin_b_ref, wg_ref, bg_ref,
               fc_w_ref, fc_b_ref, out_ref,
               adj_c, x0, hm1, hm2, a1_s, bt1_s, a2_s, bt2_s):
    s = pl.program_id(0)

    @pl.when(s < S_FA1)
    def _embed():
        rows = pl.ds(s * BE, BE)
        fb = feature_ref[...].astype(jnp.bfloat16)
        acc = jnp.dot(fb, lin_w_ref[...], preferred_element_type=jnp.float32)
        xb = jnp.maximum(acc + lin_b_ref[...], 0.0)
        x0[rows, :] = xb
        hm1[rows, :] = xb.astype(jnp.bfloat16)
        ab = jnp.dot(xb, wg_ref[:, 0:2], preferred_element_type=jnp.float32)
        a1_s[rows, :] = ab[:, 0:1] + bg_ref[0, 0]
        bt1_s[:, rows] = ab[:, 1:2].reshape(1, BE)

    @pl.when((s >= S_FA1) & (s < S_FA2))
    def _fa1():
        i = s - S_FA1
        rows = pl.ds(i * BM, BM)
        g = jnp.tanh(a1_s[rows, :] + bt1_s[...])
        hn = N // 2
        adjl = adj_l_ref[...]
        adjr = adj_r_ref[...]
        adj_c[rows, :hn] = adjl.astype(jnp.bfloat16)
        adj_c[rows, hn:] = adjr.astype(jnp.bfloat16)
        el = (adjl * g[:, :hn]).astype(jnp.bfloat16)
        er = (adjr * g[:, hn:]).astype(jnp.bfloat16)
        acc = (jnp.dot(el, hm1[pl.ds(0, hn), :], preferred_element_type=jnp.float32)
               + jnp.dot(er, hm1[pl.ds(hn, hn), :], preferred_element_type=jnp.float32))
        xb = jnp.maximum(acc, 0.0) + EPS * x0[rows, :]
        hm2[rows, :] = xb.astype(jnp.bfloat16)
        ab = jnp.dot(xb, wg_ref[:, 2:4], preferred_element_type=jnp.float32)
        a2_s[rows, :] = ab[:, 0:1] + bg_ref[0, 1]
        bt2_s[:, rows] = ab[:, 1:2].reshape(1, BM)

    @pl.when(s >= S_FA2)
    def _fa2():
        i = s - S_FA2
        rows = pl.ds(i * BQ, BQ)
        g = jnp.tanh(a2_s[rows, :] + bt2_s[...])
        e = adj_c[rows, :] * g.astype(jnp.bfloat16)
        acc = jnp.dot(e, hm2[...], preferred_element_type=jnp.float32)
        x2 = jnp.maximum(acc, 0.0) + EPS * x0[rows, :]
        o = jnp.dot(x2.astype(jnp.bfloat16), fc_w_ref[...],
                    preferred_element_type=jnp.float32) + fc_b_ref[...]
        m = jnp.max(o, axis=1, keepdims=True)
        lse = jnp.log(jnp.sum(jnp.exp(o - m), axis=1, keepdims=True))
        out_ref[...] = o - m - lse


@jax.jit
def kernel(feature, adj, lin_w, lin_b, gate_w, gate_b, fc_w, fc_b):
    hh = gate_w.shape[1] // 2
    wg = jnp.stack([gate_w[0, :hh], gate_w[0, hh:],
                    gate_w[1, :hh], gate_w[1, hh:]], axis=1)
    bg = gate_b.reshape(1, 2)

    return pl.pallas_call(
        _mega_body,
        grid=(N_STEPS,),
        in_specs=[
            pl.BlockSpec((BE, F_IN),
                         lambda s: (jnp.minimum(s, NE - 1), 0)),
            pl.BlockSpec((BM, N // 2),
                         lambda s: (jnp.where(s < S_FA1, 0,
                                              jnp.minimum(s - S_FA1, NB - 1)), 0)),
            pl.BlockSpec((BM, N // 2),
                         lambda s: (jnp.where(s < S_FA1, 0,
                                              jnp.minimum(s - S_FA1, NB - 1)), 1)),
            pl.BlockSpec((F_IN, H), lambda s: (0, 0)),
            pl.BlockSpec((1, H), lambda s: (0, 0)),
            pl.BlockSpec((H, 4), lambda s: (0, 0)),
            pl.BlockSpec((1, 2), lambda s: (0, 0)),
            pl.BlockSpec((H, C), lambda s: (0, 0)),
            pl.BlockSpec((1, C), lambda s: (0, 0)),
        ],
        out_specs=pl.BlockSpec((BQ, C),
                               lambda s: (jnp.where(s >= S_FA2, s - S_FA2, 0), 0)),
        out_shape=jax.ShapeDtypeStruct((N, C), jnp.float32),
        scratch_shapes=[
            pltpu.VMEM((N, N), jnp.bfloat16),
            pltpu.VMEM((N, H), jnp.float32),
            pltpu.VMEM((N, H), jnp.bfloat16),
            pltpu.VMEM((N, H), jnp.bfloat16),
            pltpu.VMEM((N, 1), jnp.float32),
            pltpu.VMEM((1, N), jnp.float32),
            pltpu.VMEM((N, 1), jnp.float32),
            pltpu.VMEM((1, N), jnp.float32),
        ],
    )(feature, adj, adj, lin_w.astype(jnp.bfloat16), lin_b.reshape(1, H),
      wg, bg, fc_w.astype(jnp.bfloat16), fc_b.reshape(1, C))


# speedup vs baseline: 1.0045x; 1.0045x over previous
import jax
import jax.numpy as jnp
from jax.experimental import pallas as pl
from jax.experimental.pallas import tpu as pltpu

N = 4096
F_IN = 512
H = 256
C = 64
EPS = 0.3
NB = 16
BM = N // NB
NE = 4
BE = N // NE
NQ = 8
BQ = N // NQ
S_FA1 = NE
S_FA2 = NE + NB
N_STEPS = NE + NB + NQ


def _mega_body(feature_ref, adj_ref, lin_w_ref, lin_b_ref, wg_ref, bg_ref,
               fc_w_ref, fc_b_ref, out_ref,
               adj_c, x0, hm1, hm2, a1_s, bt1_s, a2_s, bt2_s):
    s = pl.program_id(0)

    @pl.when(s < S_FA1)
    def _embed():
        rows = pl.ds(s * BE, BE)
        fb = feature_ref[...].astype(jnp.bfloat16)
        acc = jnp.dot(fb, lin_w_ref[...], preferred_element_type=jnp.float32)
        xb = jnp.maximum(acc + lin_b_ref[...], 0.0)
        x0[rows, :] = xb
        hm1[rows, :] = xb.astype(jnp.bfloat16)
        ab = jnp.dot(xb, wg_ref[:, 0:2], preferred_element_type=jnp.float32)
        a1_s[rows, :] = ab[:, 0:1] + bg_ref[0, 0]
        bt1_s[:, rows] = ab[:, 1:2].reshape(1, BE)

    @pl.when((s >= S_FA1) & (s < S_FA2))
    def _fa1():
        i = s - S_FA1
        rows = pl.ds(i * BM, BM)
        g = jnp.tanh(a1_s[rows, :] + bt1_s[...])
        adjf = adj_ref[...]
        adj_c[rows, :] = adjf.astype(jnp.bfloat16)
        e = (adjf * g).astype(jnp.bfloat16)
        acc = jnp.dot(e, hm1[...], preferred_element_type=jnp.float32)
        xb = jnp.maximum(acc, 0.0) + EPS * x0[rows, :]
        hm2[rows, :] = xb.astype(jnp.bfloat16)
        ab = jnp.dot(xb, wg_ref[:, 2:4], preferred_element_type=jnp.float32)
        a2_s[rows, :] = ab[:, 0:1] + bg_ref[0, 1]
        bt2_s[:, rows] = ab[:, 1:2].reshape(1, BM)

    @pl.when(s >= S_FA2)
    def _fa2():
        i = s - S_FA2
        rows = pl.ds(i * BQ, BQ)
        g = jnp.tanh(a2_s[rows, :] + bt2_s[...])
        e = adj_c[rows, :] * g.astype(jnp.bfloat16)
        acc = jnp.dot(e, hm2[...], preferred_element_type=jnp.float32)
        x2 = jnp.maximum(acc, 0.0) + EPS * x0[rows, :]
        o = jnp.dot(x2.astype(jnp.bfloat16), fc_w_ref[...],
                    preferred_element_type=jnp.float32) + fc_b_ref[...]
        m = jnp.max(o, axis=1, keepdims=True)
        lse = jnp.log(jnp.sum(jnp.exp(o - m), axis=1, keepdims=True))
        out_ref[...] = o - m - lse


@jax.jit
def kernel(feature, adj, lin_w, lin_b, gate_w, gate_b, fc_w, fc_b):
    hh = gate_w.shape[1] // 2
    wg = jnp.stack([gate_w[0, :hh], gate_w[0, hh:],
                    gate_w[1, :hh], gate_w[1, hh:]], axis=1)
    bg = gate_b.reshape(1, 2)

    return pl.pallas_call(
        _mega_body,
        grid=(N_STEPS,),
        in_specs=[
            pl.BlockSpec((BE, F_IN),
                         lambda s: (jnp.minimum(s, NE - 1), 0)),
            pl.BlockSpec((BM, N),
                         lambda s: (jnp.where(s < S_FA1, 0,
                                              jnp.minimum(s - S_FA1, NB - 1)), 0)),
            pl.BlockSpec((F_IN, H), lambda s: (0, 0)),
            pl.BlockSpec((1, H), lambda s: (0, 0)),
            pl.BlockSpec((H, 4), lambda s: (0, 0)),
            pl.BlockSpec((1, 2), lambda s: (0, 0)),
            pl.BlockSpec((H, C), lambda s: (0, 0)),
            pl.BlockSpec((1, C), lambda s: (0, 0)),
        ],
        out_specs=pl.BlockSpec((BQ, C),
                               lambda s: (jnp.where(s >= S_FA2, s - S_FA2, 0), 0)),
        out_shape=jax.ShapeDtypeStruct((N, C), jnp.float32),
        scratch_shapes=[
            pltpu.VMEM((N, N), jnp.bfloat16),
            pltpu.VMEM((N, H), jnp.float32),
            pltpu.VMEM((N, H), jnp.bfloat16),
            pltpu.VMEM((N, H), jnp.bfloat16),
            pltpu.VMEM((N, 1), jnp.float32),
            pltpu.VMEM((1, N), jnp.float32),
            pltpu.VMEM((N, 1), jnp.float32),
            pltpu.VMEM((1, N), jnp.float32),
        ],
    )(feature, adj, lin_w.astype(jnp.bfloat16), lin_b.reshape(1, H),
      wg, bg, fc_w.astype(jnp.bfloat16), fc_b.reshape(1, C))
